# baseline (device time: 18039 ns/iter reference)
import jax
import jax.numpy as jnp
from jax import lax
from jax.experimental import pallas as pl
from jax.experimental.pallas import tpu as pltpu


def kernel(x):
    m, n = x.shape
    half = n // 2

    def body(x_ref, out_ref, local_sem, send_sem, recv_sem):
        my_x = lax.axis_index("x")
        my_y = lax.axis_index("y")
        my_z = lax.axis_index("z")
        other_y = 1 - my_y
        peer = (my_x, other_y, my_z)

        barrier_sem = pltpu.get_barrier_semaphore()
        pl.semaphore_signal(
            barrier_sem, inc=1,
            device_id=peer, device_id_type=pl.DeviceIdType.MESH,
        )
        pl.semaphore_wait(barrier_sem, 1)

        rdma = pltpu.make_async_remote_copy(
            src_ref=x_ref.at[:, pl.ds(other_y * half, half)],
            dst_ref=out_ref.at[pl.ds(my_y * m, m)],
            send_sem=send_sem,
            recv_sem=recv_sem,
            device_id=peer,
            device_id_type=pl.DeviceIdType.MESH,
        )
        rdma.start()

        local = pltpu.make_async_copy(
            x_ref.at[:, pl.ds(my_y * half, half)],
            out_ref.at[pl.ds(my_y * m, m)],
            local_sem,
        )
        local.start()

        local.wait()
        rdma.wait()

    return pl.pallas_call(
        body,
        out_shape=jax.ShapeDtypeStruct((2 * m, half), x.dtype),
        in_specs=[pl.BlockSpec(memory_space=pl.ANY)],
        out_specs=pl.BlockSpec(memory_space=pl.ANY),
        scratch_shapes=[
            pltpu.SemaphoreType.DMA,
            pltpu.SemaphoreType.DMA,
            pltpu.SemaphoreType.DMA,
        ],
        compiler_params=pltpu.CompilerParams(collective_id=0),
    )(x)


# device time: 17705 ns/iter; 1.0189x vs baseline; 1.0189x over previous
import jax
import jax.numpy as jnp
from jax import lax
from jax.experimental import pallas as pl
from jax.experimental.pallas import tpu as pltpu

NCHUNK = 2


def kernel(x):
    m, n = x.shape
    half = n // 2
    hm = m // 2
    rpc = hm // NCHUNK

    def body(x_ref, out_ref, local_sem,
             ysend_sems, yrecv_sems, xsend_sems, xrecv_sems):
        my_x = lax.axis_index("x")
        my_y = lax.axis_index("y")
        my_z = lax.axis_index("z")
        other_y = 1 - my_y
        ypeer = (my_x, other_y, my_z)
        xpeer = (1 - my_x, my_y, my_z)

        barrier_sem = pltpu.get_barrier_semaphore()
        for nbr in (ypeer, xpeer):
            pl.semaphore_signal(
                barrier_sem, inc=1,
                device_id=nbr, device_id_type=pl.DeviceIdType.MESH,
            )
        pl.semaphore_wait(barrier_sem, 2)

        y_rdmas = []
        for c in range(NCHUNK):
            row0 = my_y * m + my_x * hm + c * rpc
            rdma = pltpu.make_async_remote_copy(
                src_ref=x_ref.at[pl.ds(my_x * hm + c * rpc, rpc),
                                 pl.ds(other_y * half, half)],
                dst_ref=out_ref.at[pl.ds(row0, rpc)],
                send_sem=ysend_sems.at[c],
                recv_sem=yrecv_sems.at[c],
                device_id=ypeer,
                device_id_type=pl.DeviceIdType.MESH,
            )
            rdma.start()
            y_rdmas.append(rdma)

        local = pltpu.make_async_copy(
            x_ref.at[:, pl.ds(my_y * half, half)],
            out_ref.at[pl.ds(my_y * m, m)],
            local_sem,
        )
        local.start()

        x_rdmas = []
        for c in range(NCHUNK):
            y_rdmas[c].wait_recv()
            row0 = other_y * m + my_x * hm + c * rpc
            fwd = pltpu.make_async_remote_copy(
                src_ref=out_ref.at[pl.ds(row0, rpc)],
                dst_ref=out_ref.at[pl.ds(row0, rpc)],
                send_sem=xsend_sems.at[c],
                recv_sem=xrecv_sems.at[c],
                device_id=xpeer,
                device_id_type=pl.DeviceIdType.MESH,
            )
            fwd.start()
            x_rdmas.append(fwd)

        for c in range(NCHUNK):
            x_rdmas[c].wait_recv()
            x_rdmas[c].wait_send()
            y_rdmas[c].wait_send()
        local.wait()

    return pl.pallas_call(
        body,
        out_shape=jax.ShapeDtypeStruct((2 * m, half), x.dtype),
        in_specs=[pl.BlockSpec(memory_space=pl.ANY)],
        out_specs=pl.BlockSpec(memory_space=pl.ANY),
        scratch_shapes=[
            pltpu.SemaphoreType.DMA,
            pltpu.SemaphoreType.DMA((NCHUNK,)),
            pltpu.SemaphoreType.DMA((NCHUNK,)),
            pltpu.SemaphoreType.DMA((NCHUNK,)),
            pltpu.SemaphoreType.DMA((NCHUNK,)),
        ],
        compiler_params=pltpu.CompilerParams(collective_id=0),
    )(x)


# device time: 17339 ns/iter; 1.0404x vs baseline; 1.0211x over previous
import jax
import jax.numpy as jnp
from jax import lax
from jax.experimental import pallas as pl
from jax.experimental.pallas import tpu as pltpu

NCHUNK = 2


def kernel(x):
    m, n = x.shape
    half = n // 2
    qm = m // 4
    rpc = qm // NCHUNK

    def body(x_ref, out_ref, local_sem,
             ysend, yrecv, xsend, xrecv, zsend, zrecv, dsend, drecv):
        my_x = lax.axis_index("x")
        my_y = lax.axis_index("y")
        my_z = lax.axis_index("z")
        other_y = 1 - my_y
        ypeer = (my_x, other_y, my_z)
        xpeer = (1 - my_x, my_y, my_z)
        zpeer = (my_x, my_y, 1 - my_z)
        dpeer = (1 - my_x, my_y, 1 - my_z)
        q = 2 * my_x + my_z

        barrier_sem = pltpu.get_barrier_semaphore()
        for nbr in (ypeer, xpeer, zpeer, dpeer):
            pl.semaphore_signal(
                barrier_sem, inc=1,
                device_id=nbr, device_id_type=pl.DeviceIdType.MESH,
            )
        pl.semaphore_wait(barrier_sem, 4)

        y_rdmas = []
        for c in range(NCHUNK):
            rdma = pltpu.make_async_remote_copy(
                src_ref=x_ref.at[pl.ds(q * qm + c * rpc, rpc),
                                 pl.ds(other_y * half, half)],
                dst_ref=out_ref.at[pl.ds(my_y * m + q * qm + c * rpc, rpc)],
                send_sem=ysend.at[c],
                recv_sem=yrecv.at[c],
                device_id=ypeer,
                device_id_type=pl.DeviceIdType.MESH,
            )
            rdma.start()
            y_rdmas.append(rdma)

        local = pltpu.make_async_copy(
            x_ref.at[:, pl.ds(my_y * half, half)],
            out_ref.at[pl.ds(my_y * m, m)],
            local_sem,
        )
        local.start()

        fwd_rdmas = []
        for c in range(NCHUNK):
            y_rdmas[c].wait_recv()
            row0 = other_y * m + q * qm + c * rpc
            for peer, ssem, rsem in (
                (xpeer, xsend, xrecv),
                (zpeer, zsend, zrecv),
                (dpeer, dsend, drecv),
            ):
                fwd = pltpu.make_async_remote_copy(
                    src_ref=out_ref.at[pl.ds(row0, rpc)],
                    dst_ref=out_ref.at[pl.ds(row0, rpc)],
                    send_sem=ssem.at[c],
                    recv_sem=rsem.at[c],
                    device_id=peer,
                    device_id_type=pl.DeviceIdType.MESH,
                )
                fwd.start()
                fwd_rdmas.append(fwd)

        for fwd in fwd_rdmas:
            fwd.wait_recv()
        for fwd in fwd_rdmas:
            fwd.wait_send()
        for c in range(NCHUNK):
            y_rdmas[c].wait_send()
        local.wait()

    return pl.pallas_call(
        body,
        out_shape=jax.ShapeDtypeStruct((2 * m, half), x.dtype),
        in_specs=[pl.BlockSpec(memory_space=pl.ANY)],
        out_specs=pl.BlockSpec(memory_space=pl.ANY),
        scratch_shapes=[
            pltpu.SemaphoreType.DMA,
            pltpu.SemaphoreType.DMA((NCHUNK,)),
            pltpu.SemaphoreType.DMA((NCHUNK,)),
            pltpu.SemaphoreType.DMA((NCHUNK,)),
            pltpu.SemaphoreType.DMA((NCHUNK,)),
            pltpu.SemaphoreType.DMA((NCHUNK,)),
            pltpu.SemaphoreType.DMA((NCHUNK,)),
            pltpu.SemaphoreType.DMA((NCHUNK,)),
            pltpu.SemaphoreType.DMA((NCHUNK,)),
        ],
        compiler_params=pltpu.CompilerParams(collective_id=0),
    )(x)


# device time: 17044 ns/iter; 1.0584x vs baseline; 1.0173x over previous
import jax
import jax.numpy as jnp
from jax import lax
from jax.experimental import pallas as pl
from jax.experimental.pallas import tpu as pltpu

NCHUNK = 4


def kernel(x):
    m, n = x.shape
    half = n // 2
    qm = m // 4
    rpc = qm // NCHUNK

    def body(x_ref, out_ref, local_sem,
             ysend, yrecv, xsend, xrecv, zsend, zrecv, dsend, drecv):
        my_x = lax.axis_index("x")
        my_y = lax.axis_index("y")
        my_z = lax.axis_index("z")
        other_y = 1 - my_y
        ypeer = (my_x, other_y, my_z)
        xpeer = (1 - my_x, my_y, my_z)
        zpeer = (my_x, my_y, 1 - my_z)
        dpeer = (1 - my_x, my_y, 1 - my_z)
        q = 2 * my_x + my_z

        local = pltpu.make_async_copy(
            x_ref.at[:, pl.ds(my_y * half, half)],
            out_ref.at[pl.ds(my_y * m, m)],
            local_sem,
        )
        local.start()

        barrier_sem = pltpu.get_barrier_semaphore()
        for nbr in (ypeer, xpeer, zpeer, dpeer):
            pl.semaphore_signal(
                barrier_sem, inc=1,
                device_id=nbr, device_id_type=pl.DeviceIdType.MESH,
            )
        pl.semaphore_wait(barrier_sem, 4)

        y_rdmas = []
        for c in range(NCHUNK):
            rdma = pltpu.make_async_remote_copy(
                src_ref=x_ref.at[pl.ds(q * qm + c * rpc, rpc),
                                 pl.ds(other_y * half, half)],
                dst_ref=out_ref.at[pl.ds(my_y * m + q * qm + c * rpc, rpc)],
                send_sem=ysend.at[c],
                recv_sem=yrecv.at[c],
                device_id=ypeer,
                device_id_type=pl.DeviceIdType.MESH,
            )
            rdma.start()
            y_rdmas.append(rdma)

        fwd_rdmas = []
        for c in range(NCHUNK):
            y_rdmas[c].wait_recv()
            row0 = other_y * m + q * qm + c * rpc
            for peer, ssem, rsem in (
                (xpeer, xsend, xrecv),
                (zpeer, zsend, zrecv),
                (dpeer, dsend, drecv),
            ):
                fwd = pltpu.make_async_remote_copy(
                    src_ref=out_ref.at[pl.ds(row0, rpc)],
                    dst_ref=out_ref.at[pl.ds(row0, rpc)],
                    send_sem=ssem.at[c],
                    recv_sem=rsem.at[c],
                    device_id=peer,
                    device_id_type=pl.DeviceIdType.MESH,
                )
                fwd.start()
                fwd_rdmas.append(fwd)

        for fwd in fwd_rdmas:
            fwd.wait_recv()
        for fwd in fwd_rdmas:
            fwd.wait_send()
        for c in range(NCHUNK):
            y_rdmas[c].wait_send()
        local.wait()

    return pl.pallas_call(
        body,
        out_shape=jax.ShapeDtypeStruct((2 * m, half), x.dtype),
        in_specs=[pl.BlockSpec(memory_space=pl.ANY)],
        out_specs=pl.BlockSpec(memory_space=pl.ANY),
        scratch_shapes=[
            pltpu.SemaphoreType.DMA,
            pltpu.SemaphoreType.DMA((NCHUNK,)),
            pltpu.SemaphoreType.DMA((NCHUNK,)),
            pltpu.SemaphoreType.DMA((NCHUNK,)),
            pltpu.SemaphoreType.DMA((NCHUNK,)),
            pltpu.SemaphoreType.DMA((NCHUNK,)),
            pltpu.SemaphoreType.DMA((NCHUNK,)),
            pltpu.SemaphoreType.DMA((NCHUNK,)),
            pltpu.SemaphoreType.DMA((NCHUNK,)),
        ],
        compiler_params=pltpu.CompilerParams(collective_id=0),
    )(x)
